# baseline (speedup 1.0000x reference)
import functools

import jax
import jax.numpy as jnp
from jax import lax
from jax.experimental import pallas as pl
from jax.experimental.pallas import tpu as pltpu
from jax.experimental.pallas import tpu_sc as plsc

NC = 2
NS = 16
NW = NC * NS
LANES = 16
CH = 128


def _zero16():
    return jnp.zeros((LANES,), jnp.float32)


def _ones16():
    return jnp.ones((LANES,), jnp.float32)


def _make_deg_kernel(NPAD, E):
    NCHT = E // CH
    NB = NCHT // NW
    R = NCHT - NB * NW
    RPT = NPAD // NS
    mesh = plsc.VectorSubcoreMesh(core_axis_name="c", subcore_axis_name="s")

    @functools.partial(
        pl.kernel,
        out_type=jax.ShapeDtypeStruct((NC, NPAD, LANES), jnp.float32),
        mesh=mesh,
        scratch_types=[
            pltpu.VMEM_SHARED((NPAD, LANES), jnp.float32),
            pltpu.VMEM((CH,), jnp.int32),
            pltpu.VMEM((CH,), jnp.int32),
            pltpu.VMEM((CH, LANES), jnp.float32),
            pltpu.VMEM((RPT, LANES), jnp.float32),
            pltpu.SemaphoreType.DMA,
            pltpu.SemaphoreType.DMA,
        ],
    )
    def deg_kernel(dst_hbm, out_hbm, acc_sh, idx_a, idx_b, ones_v, stage_v,
                   sem_a, sem_b):
        c = lax.axis_index("c")
        s = lax.axis_index("s")
        wid = s * NC + c
        start = (wid * NB + jnp.minimum(wid, R)) * CH

        @pl.loop(0, RPT)
        def _(r):
            stage_v[r] = _zero16()

        @pl.loop(0, CH)
        def _(r):
            ones_v[r] = _ones16()

        pltpu.sync_copy(stage_v, acc_sh.at[pl.ds(s * RPT, RPT)])
        plsc.subcore_barrier()

        @pl.loop(0, NB)
        def _(g):
            e0 = start + g * CH
            pltpu.sync_copy(dst_hbm.at[pl.ds(e0, CH)], idx_a)
            pltpu.sync_copy(ones_v, acc_sh.at[idx_a], add=True)

        if False:
            e0 = start + (NB - 1) * CH
            pltpu.sync_copy(dst_hbm.at[pl.ds(e0, CH)], idx_a)
            pltpu.sync_copy(ones_v, acc_sh.at[idx_a], add=True)

        @pl.when(wid < R)
        def _():
            e0 = start + NB * CH
            pltpu.sync_copy(dst_hbm.at[pl.ds(e0, CH)], idx_a)
            pltpu.sync_copy(ones_v, acc_sh.at[idx_a], add=True)

        plsc.subcore_barrier()
        pltpu.sync_copy(acc_sh.at[pl.ds(s * RPT, RPT)],
                        out_hbm.at[c].at[pl.ds(s * RPT, RPT)])

    return deg_kernel


def _make_agg_kernel(NPAD, E, D):
    NCHT = E // CH
    NB = NCHT // NW
    R = NCHT - NB * NW
    RPT = NPAD // NS
    NZC = RPT // CH
    mesh = plsc.VectorSubcoreMesh(core_axis_name="c", subcore_axis_name="s")

    @functools.partial(
        pl.kernel,
        out_type=jax.ShapeDtypeStruct((NC, NPAD, D), jnp.float32),
        mesh=mesh,
        scratch_types=[
            pltpu.VMEM_SHARED((NPAD, D), jnp.float32),
            pltpu.VMEM((CH,), jnp.int32),
            pltpu.VMEM((CH,), jnp.int32),
            pltpu.VMEM((CH,), jnp.int32),
            pltpu.VMEM((CH,), jnp.int32),
            pltpu.VMEM((CH, D), jnp.float32),
            pltpu.VMEM((CH, D), jnp.float32),
            pltpu.SemaphoreType.DMA,
            pltpu.SemaphoreType.DMA,
        ],
    )
    def agg_kernel(src_hbm, dst_hbm, xs_hbm, out_hbm,
                   acc_sh, src_a, src_b, dst_a, dst_b,
                   rows_a, rows_b, sem_a, sem_b):
        c = lax.axis_index("c")
        s = lax.axis_index("s")
        wid = s * NC + c
        start = (wid * NB + jnp.minimum(wid, R)) * CH

        @pl.loop(0, CH)
        def _(r):
            for k in range(D // LANES):
                rows_a[r, pl.ds(k * LANES, LANES)] = _zero16()

        for j in range(NZC):
            pltpu.sync_copy(rows_a, acc_sh.at[pl.ds(s * RPT + j * CH, CH)])
        plsc.subcore_barrier()

        @pl.loop(0, NB // 2)
        def _(g):
            b0 = start + (2 * g) * CH
            b1 = b0 + CH
            pltpu.sync_copy(src_hbm.at[pl.ds(b0, CH)], src_a)
            pltpu.sync_copy(src_hbm.at[pl.ds(b1, CH)], src_b)
            d_a = pltpu.async_copy(xs_hbm.at[src_a], rows_a, sem_a)
            d_b = pltpu.async_copy(xs_hbm.at[src_b], rows_b, sem_b)
            pltpu.sync_copy(dst_hbm.at[pl.ds(b0, CH)], dst_a)
            pltpu.sync_copy(dst_hbm.at[pl.ds(b1, CH)], dst_b)
            d_a.wait()
            pltpu.sync_copy(rows_a, acc_sh.at[dst_a], add=True)
            d_b.wait()
            pltpu.sync_copy(rows_b, acc_sh.at[dst_b], add=True)

        if NB % 2:
            b0 = start + (NB - 1) * CH
            pltpu.sync_copy(src_hbm.at[pl.ds(b0, CH)], src_a)
            pltpu.sync_copy(dst_hbm.at[pl.ds(b0, CH)], dst_a)
            d_a = pltpu.async_copy(xs_hbm.at[src_a], rows_a, sem_a)
            d_a.wait()
            pltpu.sync_copy(rows_a, acc_sh.at[dst_a], add=True)

        @pl.when(wid < R)
        def _():
            b0 = start + NB * CH
            pltpu.sync_copy(src_hbm.at[pl.ds(b0, CH)], src_a)
            pltpu.sync_copy(dst_hbm.at[pl.ds(b0, CH)], dst_a)
            d_a = pltpu.async_copy(xs_hbm.at[src_a], rows_a, sem_a)
            d_a.wait()
            pltpu.sync_copy(rows_a, acc_sh.at[dst_a], add=True)

        plsc.subcore_barrier()
        pltpu.sync_copy(acc_sh.at[pl.ds(s * RPT, RPT)],
                        out_hbm.at[c].at[pl.ds(s * RPT, RPT)])

    return agg_kernel


def _prep_body(dw_ref, x_ref, xs_ref, dis_ref):
    deg = dw_ref[0, :, 0:1] + dw_ref[1, :, 0:1] + 1.0
    dis = lax.rsqrt(deg)
    dis_ref[...] = dis
    xs_ref[...] = x_ref[...] * dis


def _final_body(u0_ref, u1_ref, xs_ref, dis_ref, w_ref, b_ref, pw_ref, o_ref):
    t = (u0_ref[0] + u1_ref[0] + xs_ref[...]) * dis_ref[...]
    z = jnp.dot(t, w_ref[...], preferred_element_type=jnp.float32) + b_ref[...]
    z = jnp.where(z >= 0, z, pw_ref[...] * z)
    nrm = jnp.sqrt(jnp.sum(z * z, axis=1, keepdims=True))
    o_ref[...] = z / jnp.maximum(nrm, 1e-12)


def kernel(x, edge_index, W, b, prelu_w):
    N, D = x.shape
    H = W.shape[1]
    E = edge_index.shape[1]
    src = edge_index[0].astype(jnp.int32)
    dst = edge_index[1].astype(jnp.int32)

    NPAD = -(-N // 1280) * 1280
    dw = _make_deg_kernel(NPAD, E)(dst)

    PBLK = 2000
    xs, dis = pl.pallas_call(
        _prep_body,
        grid=(N // PBLK,),
        in_specs=[
            pl.BlockSpec((NC, PBLK, LANES), lambda i: (0, i, 0)),
            pl.BlockSpec((PBLK, D), lambda i: (i, 0)),
        ],
        out_specs=[
            pl.BlockSpec((PBLK, D), lambda i: (i, 0)),
            pl.BlockSpec((PBLK, 1), lambda i: (i, 0)),
        ],
        out_shape=[
            jax.ShapeDtypeStruct((N, D), jnp.float32),
            jax.ShapeDtypeStruct((N, 1), jnp.float32),
        ],
    )(dw, x)

    u = _make_agg_kernel(NPAD, E, D)(src, dst, xs)

    BLK = 2000
    out = pl.pallas_call(
        _final_body,
        grid=(N // BLK,),
        in_specs=[
            pl.BlockSpec((1, BLK, D), lambda i: (0, i, 0)),
            pl.BlockSpec((1, BLK, D), lambda i: (1, i, 0)),
            pl.BlockSpec((BLK, D), lambda i: (i, 0)),
            pl.BlockSpec((BLK, 1), lambda i: (i, 0)),
            pl.BlockSpec((D, H), lambda i: (0, 0)),
            pl.BlockSpec((1, H), lambda i: (0, 0)),
            pl.BlockSpec((1, H), lambda i: (0, 0)),
        ],
        out_specs=pl.BlockSpec((BLK, H), lambda i: (i, 0)),
        out_shape=jax.ShapeDtypeStruct((N, H), jnp.float32),
    )(u, u, xs, dis, W, b.reshape(1, H), prelu_w.reshape(1, H))

    return out


# speedup vs baseline: 1.0078x; 1.0078x over previous
import functools

import jax
import jax.numpy as jnp
from jax import lax
from jax.experimental import pallas as pl
from jax.experimental.pallas import tpu as pltpu
from jax.experimental.pallas import tpu_sc as plsc

NC = 2
NS = 16
NW = NC * NS
LANES = 16
CH = 128


def _zero16():
    return jnp.zeros((LANES,), jnp.float32)


def _ones16():
    return jnp.ones((LANES,), jnp.float32)


def _make_deg_kernel(NPAD, E):
    NCHT = E // CH
    NB = NCHT // NW
    R = NCHT - NB * NW
    RPT = NPAD // NS
    mesh = plsc.VectorSubcoreMesh(core_axis_name="c", subcore_axis_name="s")

    @functools.partial(
        pl.kernel,
        out_type=jax.ShapeDtypeStruct((NC, NPAD, LANES), jnp.float32),
        mesh=mesh,
        scratch_types=[
            pltpu.VMEM_SHARED((NPAD, LANES), jnp.float32),
            pltpu.VMEM((CH,), jnp.int32),
            pltpu.VMEM((CH,), jnp.int32),
            pltpu.VMEM((CH, LANES), jnp.float32),
            pltpu.VMEM((RPT, LANES), jnp.float32),
            pltpu.SemaphoreType.DMA,
            pltpu.SemaphoreType.DMA,
        ],
    )
    def deg_kernel(dst_hbm, out_hbm, acc_sh, idx_a, idx_b, ones_v, stage_v,
                   sem_a, sem_b):
        c = lax.axis_index("c")
        s = lax.axis_index("s")
        wid = s * NC + c
        start = (wid * NB + jnp.minimum(wid, R)) * CH

        @pl.loop(0, RPT)
        def _(r):
            stage_v[r] = _zero16()

        @pl.loop(0, CH)
        def _(r):
            ones_v[r] = _ones16()

        pltpu.sync_copy(stage_v, acc_sh.at[pl.ds(s * RPT, RPT)])
        plsc.subcore_barrier()

        @pl.loop(0, NB)
        def _(g):
            e0 = start + g * CH
            pltpu.sync_copy(dst_hbm.at[pl.ds(e0, CH)], idx_a)
            pltpu.sync_copy(ones_v, acc_sh.at[idx_a], add=True)

        if False:
            e0 = start + (NB - 1) * CH
            pltpu.sync_copy(dst_hbm.at[pl.ds(e0, CH)], idx_a)
            pltpu.sync_copy(ones_v, acc_sh.at[idx_a], add=True)

        @pl.when(wid < R)
        def _():
            e0 = start + NB * CH
            pltpu.sync_copy(dst_hbm.at[pl.ds(e0, CH)], idx_a)
            pltpu.sync_copy(ones_v, acc_sh.at[idx_a], add=True)

        plsc.subcore_barrier()
        pltpu.sync_copy(acc_sh.at[pl.ds(s * RPT, RPT)],
                        out_hbm.at[c].at[pl.ds(s * RPT, RPT)])

    return deg_kernel


def _make_agg_kernel(NPAD, E, D):
    NCHT = E // CH
    NB = NCHT // NW
    R = NCHT - NB * NW
    RPT = NPAD // NS
    NZC = RPT // CH
    mesh = plsc.VectorSubcoreMesh(core_axis_name="c", subcore_axis_name="s")

    @functools.partial(
        pl.kernel,
        out_type=jax.ShapeDtypeStruct((NC, NPAD, D), jnp.float32),
        mesh=mesh,
        scratch_types=[
            pltpu.VMEM_SHARED((NPAD, D), jnp.float32),
            pltpu.VMEM((CH,), jnp.int32),
            pltpu.VMEM((CH,), jnp.int32),
            pltpu.VMEM((CH,), jnp.int32),
            pltpu.VMEM((CH,), jnp.int32),
            pltpu.VMEM((CH, D), jnp.float32),
            pltpu.VMEM((CH, D), jnp.float32),
            pltpu.SemaphoreType.DMA,
            pltpu.SemaphoreType.DMA,
            pltpu.SemaphoreType.DMA,
            pltpu.SemaphoreType.DMA,
        ],
    )
    def agg_kernel(src_hbm, dst_hbm, xs_hbm, out_hbm,
                   acc_sh, src_a, src_b, dst_a, dst_b,
                   rows_a, rows_b, sem_a, sem_b, sem_c, sem_d):
        c = lax.axis_index("c")
        s = lax.axis_index("s")
        wid = s * NC + c
        start = (wid * NB + jnp.minimum(wid, R)) * CH

        @pl.loop(0, CH)
        def _(r):
            for k in range(D // LANES):
                rows_a[r, pl.ds(k * LANES, LANES)] = _zero16()

        for j in range(NZC):
            pltpu.sync_copy(rows_a, acc_sh.at[pl.ds(s * RPT + j * CH, CH)])
        plsc.subcore_barrier()

        @pl.loop(0, NB // 2)
        def _(g):
            b0 = start + (2 * g) * CH
            b1 = b0 + CH
            pltpu.sync_copy(src_hbm.at[pl.ds(b0, CH)], src_a)
            pltpu.sync_copy(src_hbm.at[pl.ds(b1, CH)], src_b)
            d_a = pltpu.async_copy(xs_hbm.at[src_a], rows_a, sem_a)
            d_b = pltpu.async_copy(xs_hbm.at[src_b], rows_b, sem_b)
            pltpu.sync_copy(dst_hbm.at[pl.ds(b0, CH)], dst_a)
            pltpu.sync_copy(dst_hbm.at[pl.ds(b1, CH)], dst_b)
            d_a.wait()
            e_a = pltpu.async_copy(rows_a, acc_sh.at[dst_a], sem_c, add=True)
            d_b.wait()
            e_b = pltpu.async_copy(rows_b, acc_sh.at[dst_b], sem_d, add=True)
            e_a.wait()
            e_b.wait()

        if NB % 2:
            b0 = start + (NB - 1) * CH
            pltpu.sync_copy(src_hbm.at[pl.ds(b0, CH)], src_a)
            pltpu.sync_copy(dst_hbm.at[pl.ds(b0, CH)], dst_a)
            d_a = pltpu.async_copy(xs_hbm.at[src_a], rows_a, sem_a)
            d_a.wait()
            pltpu.sync_copy(rows_a, acc_sh.at[dst_a], add=True)

        @pl.when(wid < R)
        def _():
            b0 = start + NB * CH
            pltpu.sync_copy(src_hbm.at[pl.ds(b0, CH)], src_a)
            pltpu.sync_copy(dst_hbm.at[pl.ds(b0, CH)], dst_a)
            d_a = pltpu.async_copy(xs_hbm.at[src_a], rows_a, sem_a)
            d_a.wait()
            pltpu.sync_copy(rows_a, acc_sh.at[dst_a], add=True)

        plsc.subcore_barrier()
        pltpu.sync_copy(acc_sh.at[pl.ds(s * RPT, RPT)],
                        out_hbm.at[c].at[pl.ds(s * RPT, RPT)])

    return agg_kernel


def _prep_body(dw_ref, x_ref, xs_ref, dis_ref):
    deg = dw_ref[0, :, 0:1] + dw_ref[1, :, 0:1] + 1.0
    dis = lax.rsqrt(deg)
    dis_ref[...] = dis
    xs_ref[...] = x_ref[...] * dis


def _final_body(u0_ref, u1_ref, xs_ref, dis_ref, w_ref, b_ref, pw_ref, o_ref):
    t = (u0_ref[0] + u1_ref[0] + xs_ref[...]) * dis_ref[...]
    z = jnp.dot(t, w_ref[...], preferred_element_type=jnp.float32) + b_ref[...]
    z = jnp.where(z >= 0, z, pw_ref[...] * z)
    nrm = jnp.sqrt(jnp.sum(z * z, axis=1, keepdims=True))
    o_ref[...] = z / jnp.maximum(nrm, 1e-12)


def kernel(x, edge_index, W, b, prelu_w):
    N, D = x.shape
    H = W.shape[1]
    E = edge_index.shape[1]
    src = edge_index[0].astype(jnp.int32)
    dst = edge_index[1].astype(jnp.int32)

    NPAD = -(-N // 1280) * 1280
    dw = _make_deg_kernel(NPAD, E)(dst)

    PBLK = 2000
    xs, dis = pl.pallas_call(
        _prep_body,
        grid=(N // PBLK,),
        in_specs=[
            pl.BlockSpec((NC, PBLK, LANES), lambda i: (0, i, 0)),
            pl.BlockSpec((PBLK, D), lambda i: (i, 0)),
        ],
        out_specs=[
            pl.BlockSpec((PBLK, D), lambda i: (i, 0)),
            pl.BlockSpec((PBLK, 1), lambda i: (i, 0)),
        ],
        out_shape=[
            jax.ShapeDtypeStruct((N, D), jnp.float32),
            jax.ShapeDtypeStruct((N, 1), jnp.float32),
        ],
    )(dw, x)

    u = _make_agg_kernel(NPAD, E, D)(src, dst, xs)

    BLK = 2000
    out = pl.pallas_call(
        _final_body,
        grid=(N // BLK,),
        in_specs=[
            pl.BlockSpec((1, BLK, D), lambda i: (0, i, 0)),
            pl.BlockSpec((1, BLK, D), lambda i: (1, i, 0)),
            pl.BlockSpec((BLK, D), lambda i: (i, 0)),
            pl.BlockSpec((BLK, 1), lambda i: (i, 0)),
            pl.BlockSpec((D, H), lambda i: (0, 0)),
            pl.BlockSpec((1, H), lambda i: (0, 0)),
            pl.BlockSpec((1, H), lambda i: (0, 0)),
        ],
        out_specs=pl.BlockSpec((BLK, H), lambda i: (i, 0)),
        out_shape=jax.ShapeDtypeStruct((N, H), jnp.float32),
    )(u, u, xs, dis, W, b.reshape(1, H), prelu_w.reshape(1, H))

    return out


# speedup vs baseline: 1.0833x; 1.0750x over previous
import functools

import jax
import jax.numpy as jnp
from jax import lax
from jax.experimental import pallas as pl
from jax.experimental.pallas import tpu as pltpu
from jax.experimental.pallas import tpu_sc as plsc

NC = 2
NS = 16
NW = NC * NS
LANES = 16
CH = 128


def _zero16():
    return jnp.zeros((LANES,), jnp.float32)


def _ones16():
    return jnp.ones((LANES,), jnp.float32)


def _make_deg_kernel(NPAD, E):
    NCHT = E // CH
    NB = NCHT // NW
    R = NCHT - NB * NW
    RPT = NPAD // NS
    mesh = plsc.VectorSubcoreMesh(core_axis_name="c", subcore_axis_name="s")

    @functools.partial(
        pl.kernel,
        out_type=jax.ShapeDtypeStruct((NC, NPAD, LANES), jnp.float32),
        mesh=mesh,
        scratch_types=[
            pltpu.VMEM_SHARED((NPAD, LANES), jnp.float32),
            pltpu.VMEM((CH,), jnp.int32),
            pltpu.VMEM((CH,), jnp.int32),
            pltpu.VMEM((CH, LANES), jnp.float32),
            pltpu.VMEM((RPT, LANES), jnp.float32),
            pltpu.SemaphoreType.DMA,
            pltpu.SemaphoreType.DMA,
            pltpu.SemaphoreType.DMA,
            pltpu.SemaphoreType.DMA,
        ],
    )
    def deg_kernel(dst_hbm, out_hbm, acc_sh, idx_a, idx_b, ones_v, stage_v,
                   sem_a, sem_b, sem_c, sem_d):
        c = lax.axis_index("c")
        s = lax.axis_index("s")
        wid = s * NC + c
        start = (wid * NB + jnp.minimum(wid, R)) * CH

        @pl.loop(0, RPT)
        def _(r):
            stage_v[r] = _zero16()

        @pl.loop(0, CH)
        def _(r):
            ones_v[r] = _ones16()

        pltpu.sync_copy(stage_v, acc_sh.at[pl.ds(s * RPT, RPT)])
        plsc.subcore_barrier()

        @pl.loop(0, NB // 2)
        def _(g):
            e0 = start + (2 * g) * CH
            d_a = pltpu.async_copy(dst_hbm.at[pl.ds(e0, CH)], idx_a, sem_a)
            d_b = pltpu.async_copy(dst_hbm.at[pl.ds(e0 + CH, CH)], idx_b,
                                   sem_b)
            d_a.wait()
            e_a = pltpu.async_copy(ones_v, acc_sh.at[idx_a], sem_c, add=True)
            d_b.wait()
            e_b = pltpu.async_copy(ones_v, acc_sh.at[idx_b], sem_d, add=True)
            e_a.wait()
            e_b.wait()

        if NB % 2:
            e0 = start + (NB - 1) * CH
            pltpu.sync_copy(dst_hbm.at[pl.ds(e0, CH)], idx_a)
            pltpu.sync_copy(ones_v, acc_sh.at[idx_a], add=True)

        @pl.when(wid < R)
        def _():
            e0 = start + NB * CH
            pltpu.sync_copy(dst_hbm.at[pl.ds(e0, CH)], idx_a)
            pltpu.sync_copy(ones_v, acc_sh.at[idx_a], add=True)

        plsc.subcore_barrier()
        pltpu.sync_copy(acc_sh.at[pl.ds(s * RPT, RPT)],
                        out_hbm.at[c].at[pl.ds(s * RPT, RPT)])

    return deg_kernel


def _make_agg_kernel(NPAD, E, D):
    NCHT = E // CH
    NB = NCHT // NW
    R = NCHT - NB * NW
    RPT = NPAD // NS
    NZC = RPT // CH
    mesh = plsc.VectorSubcoreMesh(core_axis_name="c", subcore_axis_name="s")

    @functools.partial(
        pl.kernel,
        out_type=jax.ShapeDtypeStruct((NC, NPAD, D), jnp.float32),
        mesh=mesh,
        scratch_types=[
            pltpu.VMEM_SHARED((NPAD, D), jnp.float32),
            pltpu.VMEM((CH,), jnp.int32),
            pltpu.VMEM((CH,), jnp.int32),
            pltpu.VMEM((CH,), jnp.int32),
            pltpu.VMEM((CH,), jnp.int32),
            pltpu.VMEM((CH, D), jnp.float32),
            pltpu.VMEM((CH, D), jnp.float32),
            pltpu.SemaphoreType.DMA,
            pltpu.SemaphoreType.DMA,
            pltpu.SemaphoreType.DMA,
            pltpu.SemaphoreType.DMA,
        ],
    )
    def agg_kernel(src_hbm, dst_hbm, xs_hbm, out_hbm,
                   acc_sh, src_a, src_b, dst_a, dst_b,
                   rows_a, rows_b, sem_a, sem_b, sem_c, sem_d):
        c = lax.axis_index("c")
        s = lax.axis_index("s")
        wid = s * NC + c
        start = (wid * NB + jnp.minimum(wid, R)) * CH

        @pl.loop(0, CH)
        def _(r):
            for k in range(D // LANES):
                rows_a[r, pl.ds(k * LANES, LANES)] = _zero16()

        for j in range(NZC):
            pltpu.sync_copy(rows_a, acc_sh.at[pl.ds(s * RPT + j * CH, CH)])
        plsc.subcore_barrier()

        @pl.loop(0, NB // 2)
        def _(g):
            b0 = start + (2 * g) * CH
            b1 = b0 + CH
            pltpu.sync_copy(src_hbm.at[pl.ds(b0, CH)], src_a)
            pltpu.sync_copy(src_hbm.at[pl.ds(b1, CH)], src_b)
            d_a = pltpu.async_copy(xs_hbm.at[src_a], rows_a, sem_a)
            d_b = pltpu.async_copy(xs_hbm.at[src_b], rows_b, sem_b)
            pltpu.sync_copy(dst_hbm.at[pl.ds(b0, CH)], dst_a)
            pltpu.sync_copy(dst_hbm.at[pl.ds(b1, CH)], dst_b)
            d_a.wait()
            e_a = pltpu.async_copy(rows_a, acc_sh.at[dst_a], sem_c, add=True)
            d_b.wait()
            e_b = pltpu.async_copy(rows_b, acc_sh.at[dst_b], sem_d, add=True)
            e_a.wait()
            e_b.wait()

        if NB % 2:
            b0 = start + (NB - 1) * CH
            pltpu.sync_copy(src_hbm.at[pl.ds(b0, CH)], src_a)
            pltpu.sync_copy(dst_hbm.at[pl.ds(b0, CH)], dst_a)
            d_a = pltpu.async_copy(xs_hbm.at[src_a], rows_a, sem_a)
            d_a.wait()
            pltpu.sync_copy(rows_a, acc_sh.at[dst_a], add=True)

        @pl.when(wid < R)
        def _():
            b0 = start + NB * CH
            pltpu.sync_copy(src_hbm.at[pl.ds(b0, CH)], src_a)
            pltpu.sync_copy(dst_hbm.at[pl.ds(b0, CH)], dst_a)
            d_a = pltpu.async_copy(xs_hbm.at[src_a], rows_a, sem_a)
            d_a.wait()
            pltpu.sync_copy(rows_a, acc_sh.at[dst_a], add=True)

        plsc.subcore_barrier()
        pltpu.sync_copy(acc_sh.at[pl.ds(s * RPT, RPT)],
                        out_hbm.at[c].at[pl.ds(s * RPT, RPT)])

    return agg_kernel


def _prep_body(dw_ref, x_ref, xs_ref, dis_ref):
    deg = dw_ref[0, :, 0:1] + dw_ref[1, :, 0:1] + 1.0
    dis = lax.rsqrt(deg)
    dis_ref[...] = dis
    xs_ref[...] = x_ref[...] * dis


def _final_body(u0_ref, u1_ref, xs_ref, dis_ref, w_ref, b_ref, pw_ref, o_ref):
    t = (u0_ref[0] + u1_ref[0] + xs_ref[...]) * dis_ref[...]
    z = jnp.dot(t, w_ref[...], preferred_element_type=jnp.float32) + b_ref[...]
    z = jnp.where(z >= 0, z, pw_ref[...] * z)
    nrm = jnp.sqrt(jnp.sum(z * z, axis=1, keepdims=True))
    o_ref[...] = z / jnp.maximum(nrm, 1e-12)


def kernel(x, edge_index, W, b, prelu_w):
    N, D = x.shape
    H = W.shape[1]
    E = edge_index.shape[1]
    src = edge_index[0].astype(jnp.int32)
    dst = edge_index[1].astype(jnp.int32)

    NPAD = -(-N // 1280) * 1280
    dw = _make_deg_kernel(NPAD, E)(dst)

    PBLK = 2000
    xs, dis = pl.pallas_call(
        _prep_body,
        grid=(N // PBLK,),
        in_specs=[
            pl.BlockSpec((NC, PBLK, LANES), lambda i: (0, i, 0)),
            pl.BlockSpec((PBLK, D), lambda i: (i, 0)),
        ],
        out_specs=[
            pl.BlockSpec((PBLK, D), lambda i: (i, 0)),
            pl.BlockSpec((PBLK, 1), lambda i: (i, 0)),
        ],
        out_shape=[
            jax.ShapeDtypeStruct((N, D), jnp.float32),
            jax.ShapeDtypeStruct((N, 1), jnp.float32),
        ],
    )(dw, x)

    u = _make_agg_kernel(NPAD, E, D)(src, dst, xs)

    BLK = 2000
    out = pl.pallas_call(
        _final_body,
        grid=(N // BLK,),
        in_specs=[
            pl.BlockSpec((1, BLK, D), lambda i: (0, i, 0)),
            pl.BlockSpec((1, BLK, D), lambda i: (1, i, 0)),
            pl.BlockSpec((BLK, D), lambda i: (i, 0)),
            pl.BlockSpec((BLK, 1), lambda i: (i, 0)),
            pl.BlockSpec((D, H), lambda i: (0, 0)),
            pl.BlockSpec((1, H), lambda i: (0, 0)),
            pl.BlockSpec((1, H), lambda i: (0, 0)),
        ],
        out_specs=pl.BlockSpec((BLK, H), lambda i: (i, 0)),
        out_shape=jax.ShapeDtypeStruct((N, H), jnp.float32),
    )(u, u, xs, dis, W, b.reshape(1, H), prelu_w.reshape(1, H))

    return out


# speedup vs baseline: 1.1591x; 1.0699x over previous
import functools

import jax
import jax.numpy as jnp
from jax import lax
from jax.experimental import pallas as pl
from jax.experimental.pallas import tpu as pltpu
from jax.experimental.pallas import tpu_sc as plsc

NC = 2
NS = 16
NW = NC * NS
LANES = 16
CH = 128


def _zero16():
    return jnp.zeros((LANES,), jnp.float32)


def _ones16():
    return jnp.ones((LANES,), jnp.float32)


def _make_deg_kernel(NPAD, E):
    NCHT = E // CH
    NB = NCHT // NW
    R = NCHT - NB * NW
    RPT = NPAD // NS
    mesh = plsc.VectorSubcoreMesh(core_axis_name="c", subcore_axis_name="s")

    @functools.partial(
        pl.kernel,
        out_type=jax.ShapeDtypeStruct((NC, NPAD, LANES), jnp.float32),
        mesh=mesh,
        scratch_types=[
            pltpu.VMEM_SHARED((NPAD, LANES), jnp.float32),
            pltpu.VMEM((CH,), jnp.int32),
            pltpu.VMEM((CH,), jnp.int32),
            pltpu.VMEM((CH, LANES), jnp.float32),
            pltpu.VMEM((RPT, LANES), jnp.float32),
            pltpu.SemaphoreType.DMA,
            pltpu.SemaphoreType.DMA,
            pltpu.SemaphoreType.DMA,
            pltpu.SemaphoreType.DMA,
        ],
    )
    def deg_kernel(dst_hbm, out_hbm, acc_sh, idx_a, idx_b, ones_v, stage_v,
                   sem_a, sem_b, sem_c, sem_d):
        c = lax.axis_index("c")
        s = lax.axis_index("s")
        wid = s * NC + c
        start = (wid * NB + jnp.minimum(wid, R)) * CH

        @pl.loop(0, RPT)
        def _(r):
            stage_v[r] = _zero16()

        @pl.loop(0, CH)
        def _(r):
            ones_v[r] = _ones16()

        pltpu.sync_copy(stage_v, acc_sh.at[pl.ds(s * RPT, RPT)])
        plsc.subcore_barrier()

        @pl.loop(0, NB // 2)
        def _(g):
            e0 = start + (2 * g) * CH
            d_a = pltpu.async_copy(dst_hbm.at[pl.ds(e0, CH)], idx_a, sem_a)
            d_b = pltpu.async_copy(dst_hbm.at[pl.ds(e0 + CH, CH)], idx_b,
                                   sem_b)
            d_a.wait()
            e_a = pltpu.async_copy(ones_v, acc_sh.at[idx_a], sem_c, add=True)
            d_b.wait()
            e_b = pltpu.async_copy(ones_v, acc_sh.at[idx_b], sem_d, add=True)
            e_a.wait()
            e_b.wait()

        if NB % 2:
            e0 = start + (NB - 1) * CH
            pltpu.sync_copy(dst_hbm.at[pl.ds(e0, CH)], idx_a)
            pltpu.sync_copy(ones_v, acc_sh.at[idx_a], add=True)

        @pl.when(wid < R)
        def _():
            e0 = start + NB * CH
            pltpu.sync_copy(dst_hbm.at[pl.ds(e0, CH)], idx_a)
            pltpu.sync_copy(ones_v, acc_sh.at[idx_a], add=True)

        plsc.subcore_barrier()
        pltpu.sync_copy(acc_sh.at[pl.ds(s * RPT, RPT)],
                        out_hbm.at[c].at[pl.ds(s * RPT, RPT)])

    return deg_kernel


def _make_agg_kernel(NPAD, E, D):
    NCHT = E // CH
    NB = NCHT // NW
    R = NCHT - NB * NW
    RPT = NPAD // NS
    NZC = RPT // CH
    mesh = plsc.VectorSubcoreMesh(core_axis_name="c", subcore_axis_name="s")

    @functools.partial(
        pl.kernel,
        out_type=jax.ShapeDtypeStruct((NC, NPAD, D), jnp.float32),
        mesh=mesh,
        scratch_types=[
            pltpu.VMEM_SHARED((NPAD, D), jnp.float32),
            pltpu.VMEM((CH,), jnp.int32),
            pltpu.VMEM((CH,), jnp.int32),
            pltpu.VMEM((CH,), jnp.int32),
            pltpu.VMEM((CH,), jnp.int32),
            pltpu.VMEM((CH, D), jnp.float32),
            pltpu.VMEM((CH, D), jnp.float32),
            pltpu.SemaphoreType.DMA,
            pltpu.SemaphoreType.DMA,
            pltpu.SemaphoreType.DMA,
            pltpu.SemaphoreType.DMA,
            pltpu.SemaphoreType.DMA,
            pltpu.SemaphoreType.DMA,
            pltpu.SemaphoreType.DMA,
            pltpu.SemaphoreType.DMA,
        ],
    )
    def agg_kernel(src_hbm, dst_hbm, xs_hbm, out_hbm,
                   acc_sh, src_a, src_b, dst_a, dst_b,
                   rows_a, rows_b, sem_a, sem_b, sem_c, sem_d,
                   sem_e, sem_f, sem_g, sem_h):
        c = lax.axis_index("c")
        s = lax.axis_index("s")
        wid = s * NC + c
        start = (wid * NB + jnp.minimum(wid, R)) * CH

        @pl.loop(0, CH)
        def _(r):
            for k in range(D // LANES):
                rows_a[r, pl.ds(k * LANES, LANES)] = _zero16()

        for j in range(NZC):
            pltpu.sync_copy(rows_a, acc_sh.at[pl.ds(s * RPT + j * CH, CH)])
        plsc.subcore_barrier()

        @pl.loop(0, NB // 2)
        def _(g):
            b0 = start + (2 * g) * CH
            b1 = b0 + CH
            l_sa = pltpu.async_copy(src_hbm.at[pl.ds(b0, CH)], src_a, sem_e)
            l_sb = pltpu.async_copy(src_hbm.at[pl.ds(b1, CH)], src_b, sem_f)
            l_da = pltpu.async_copy(dst_hbm.at[pl.ds(b0, CH)], dst_a, sem_g)
            l_db = pltpu.async_copy(dst_hbm.at[pl.ds(b1, CH)], dst_b, sem_h)
            l_sa.wait()
            d_a = pltpu.async_copy(xs_hbm.at[src_a], rows_a, sem_a)
            l_sb.wait()
            d_b = pltpu.async_copy(xs_hbm.at[src_b], rows_b, sem_b)
            l_da.wait()
            l_db.wait()
            d_a.wait()
            e_a = pltpu.async_copy(rows_a, acc_sh.at[dst_a], sem_c, add=True)
            d_b.wait()
            e_b = pltpu.async_copy(rows_b, acc_sh.at[dst_b], sem_d, add=True)
            e_a.wait()
            e_b.wait()

        if NB % 2:
            b0 = start + (NB - 1) * CH
            pltpu.sync_copy(src_hbm.at[pl.ds(b0, CH)], src_a)
            pltpu.sync_copy(dst_hbm.at[pl.ds(b0, CH)], dst_a)
            d_a = pltpu.async_copy(xs_hbm.at[src_a], rows_a, sem_a)
            d_a.wait()
            pltpu.sync_copy(rows_a, acc_sh.at[dst_a], add=True)

        @pl.when(wid < R)
        def _():
            b0 = start + NB * CH
            pltpu.sync_copy(src_hbm.at[pl.ds(b0, CH)], src_a)
            pltpu.sync_copy(dst_hbm.at[pl.ds(b0, CH)], dst_a)
            d_a = pltpu.async_copy(xs_hbm.at[src_a], rows_a, sem_a)
            d_a.wait()
            pltpu.sync_copy(rows_a, acc_sh.at[dst_a], add=True)

        plsc.subcore_barrier()
        pltpu.sync_copy(acc_sh.at[pl.ds(s * RPT, RPT)],
                        out_hbm.at[c].at[pl.ds(s * RPT, RPT)])

    return agg_kernel


def _prep_body(dw_ref, x_ref, xs_ref, dis_ref):
    deg = dw_ref[0, :, 0:1] + dw_ref[1, :, 0:1] + 1.0
    dis = lax.rsqrt(deg)
    dis_ref[...] = dis
    xs_ref[...] = x_ref[...] * dis


def _final_body(u0_ref, u1_ref, xs_ref, dis_ref, w_ref, b_ref, pw_ref, o_ref):
    t = (u0_ref[0] + u1_ref[0] + xs_ref[...]) * dis_ref[...]
    z = jnp.dot(t, w_ref[...], preferred_element_type=jnp.float32) + b_ref[...]
    z = jnp.where(z >= 0, z, pw_ref[...] * z)
    nrm = jnp.sqrt(jnp.sum(z * z, axis=1, keepdims=True))
    o_ref[...] = z / jnp.maximum(nrm, 1e-12)


def kernel(x, edge_index, W, b, prelu_w):
    N, D = x.shape
    H = W.shape[1]
    E = edge_index.shape[1]
    src = edge_index[0].astype(jnp.int32)
    dst = edge_index[1].astype(jnp.int32)

    NPAD = -(-N // 1280) * 1280
    dw = _make_deg_kernel(NPAD, E)(dst)

    PBLK = 2000
    xs, dis = pl.pallas_call(
        _prep_body,
        grid=(N // PBLK,),
        in_specs=[
            pl.BlockSpec((NC, PBLK, LANES), lambda i: (0, i, 0)),
            pl.BlockSpec((PBLK, D), lambda i: (i, 0)),
        ],
        out_specs=[
            pl.BlockSpec((PBLK, D), lambda i: (i, 0)),
            pl.BlockSpec((PBLK, 1), lambda i: (i, 0)),
        ],
        out_shape=[
            jax.ShapeDtypeStruct((N, D), jnp.float32),
            jax.ShapeDtypeStruct((N, 1), jnp.float32),
        ],
    )(dw, x)

    u = _make_agg_kernel(NPAD, E, D)(src, dst, xs)

    BLK = 2000
    out = pl.pallas_call(
        _final_body,
        grid=(N // BLK,),
        in_specs=[
            pl.BlockSpec((1, BLK, D), lambda i: (0, i, 0)),
            pl.BlockSpec((1, BLK, D), lambda i: (1, i, 0)),
            pl.BlockSpec((BLK, D), lambda i: (i, 0)),
            pl.BlockSpec((BLK, 1), lambda i: (i, 0)),
            pl.BlockSpec((D, H), lambda i: (0, 0)),
            pl.BlockSpec((1, H), lambda i: (0, 0)),
            pl.BlockSpec((1, H), lambda i: (0, 0)),
        ],
        out_specs=pl.BlockSpec((BLK, H), lambda i: (i, 0)),
        out_shape=jax.ShapeDtypeStruct((N, H), jnp.float32),
    )(u, u, xs, dis, W, b.reshape(1, H), prelu_w.reshape(1, H))

    return out
